# baseline (speedup 1.0000x reference)
import functools

import jax
import jax.numpy as jnp
from jax import lax
from jax.experimental import pallas as pl
from jax.experimental.pallas import tpu as pltpu
from jax.experimental.pallas import tpu_sc as plsc

N = 10000
E = 160000
H = 256

NC = 2
NS = 16
L = 16
NW = NC * NS

NCK = 2
EC = E // NCK
CH = 128
NCHUNK = EC // CH
CPW_G = -(-NCHUNK // NW)

@functools.cache
def _make_sc_gather():
    mesh = plsc.VectorSubcoreMesh(core_axis_name="c", subcore_axis_name="s")
    return functools.partial(
        pl.kernel,
        out_type=[jax.ShapeDtypeStruct((EC, H), jnp.float32)] * 3,
        mesh=mesh,
        scratch_types=[
            pltpu.VMEM((CH,), jnp.int32),
            pltpu.VMEM((CH,), jnp.int32),
            pltpu.VMEM((CH, H), jnp.float32),
            pltpu.VMEM((CH, H), jnp.float32),
            pltpu.VMEM((CH, H), jnp.float32),
            pltpu.SemaphoreType.DMA,
        ],
    )(_sc_gather_body)


def _sc_gather_body(vx_hbm, vx2_hbm, src_hbm, dst_hbm, oS, oD, o2,
                    si_v, di_v, bS, bD, b2, sem):
    wid = lax.axis_index("s") * NC + lax.axis_index("c")

    @pl.loop(0, CPW_G)
    def _(j):
        chunk = wid + j * NW

        @pl.when(chunk < NCHUNK)
        def _():
            base = chunk * CH
            pltpu.sync_copy(src_hbm.at[pl.ds(base, CH)], si_v)
            pltpu.sync_copy(dst_hbm.at[pl.ds(base, CH)], di_v)
            cS = pltpu.async_copy(vx_hbm.at[si_v], bS, sem)
            cD = pltpu.async_copy(vx_hbm.at[di_v], bD, sem)
            c2 = pltpu.async_copy(vx2_hbm.at[si_v], b2, sem)
            cS.wait()
            cD.wait()
            c2.wait()
            pltpu.sync_copy(bS, oS.at[pl.ds(base, CH)])
            pltpu.sync_copy(bD, oD.at[pl.ds(base, CH)])
            pltpu.sync_copy(b2, o2.at[pl.ds(base, CH)])


def _node_mm_body(x_ref, wv_ref, bv_ref, wu_ref, bu_ref, w2_ref, b2_ref,
                  vx_ref, ux_ref, vx2_ref):
    xb = x_ref[...]
    vx_ref[...] = jnp.dot(xb, wv_ref[...],
                          preferred_element_type=jnp.float32) + bv_ref[...]
    ux_ref[...] = jnp.dot(xb, wu_ref[...],
                          preferred_element_type=jnp.float32) + bu_ref[...]
    vx2_ref[...] = jnp.dot(xb, w2_ref[...],
                           preferred_element_type=jnp.float32) + b2_ref[...]


def _edge_body(dst_ref, e_ref, vxs_ref, vxd_ref, vx2s_ref, w_ref, b_ref,
               g_ref, be_ref, enew_ref, agg_ref, agg2_ref, gn_scr):
    eb = e_ref[...]
    et = (jnp.dot(eb, w_ref[...], preferred_element_type=jnp.float32)
          + b_ref[...] + vxs_ref[...] + vxd_ref[...])
    gate = jax.nn.sigmoid(et)
    gn_scr[...] = vx2s_ref[...] * gate
    m = jnp.mean(et, axis=-1, keepdims=True)
    var = jnp.mean((et - m) * (et - m), axis=-1, keepdims=True)
    ln = (et - m) * lax.rsqrt(var + 1e-5) * g_ref[...] + be_ref[...]
    enew_ref[...] = eb + jnp.maximum(ln, 0.0)

    @pl.when(pl.program_id(0) == 0)
    def _():
        agg_ref[...] = jnp.zeros_like(agg_ref)
        agg2_ref[...] = jnp.zeros_like(agg2_ref)

    def _acc(k2, _):
        k = k2 * 2
        d0 = dst_ref[0, 0, k]
        d1 = dst_ref[0, 0, k + 1]
        agg_ref[pl.ds(d0, 1), :] += gn_scr[pl.ds(k, 1), :]
        agg2_ref[pl.ds(d1, 1), :] += gn_scr[pl.ds(k + 1, 1), :]
        return 0

    lax.fori_loop(0, BE // 2, _acc, 0, unroll=4)


def _node_fin_body(x_ref, ux_ref, a0_ref, a1_ref, a2_ref, a3_ref,
                   g_ref, b_ref, xnew_ref):
    xt = (ux_ref[...] + (a0_ref[...] + a1_ref[...])
          + (a2_ref[...] + a3_ref[...]))
    m = jnp.mean(xt, axis=-1, keepdims=True)
    var = jnp.mean((xt - m) * (xt - m), axis=-1, keepdims=True)
    ln = (xt - m) * lax.rsqrt(var + 1e-5) * g_ref[...] + b_ref[...]
    xnew_ref[...] = x_ref[...] + jnp.maximum(ln, 0.0)


def _row_spec(bm):
    return pl.BlockSpec((bm, H), lambda i: (i, 0))


def _full_spec(shape):
    return pl.BlockSpec(shape, lambda i: (0,) * len(shape))


BN = 2000
BE = 2000


def kernel(x, e, edge_index, eU_W, eU_b, eV_W, eV_b, nU_W, nU_b, nV_W, nV_b,
           ln_e_g, ln_e_b, ln_n_g, ln_n_b):
    f32 = jnp.float32
    src = edge_index[0]
    dst = edge_index[1]

    node_mm = pl.pallas_call(
        _node_mm_body,
        grid=(N // BN,),
        in_specs=[_row_spec(BN)] + [_full_spec((H, H)), _full_spec((1, H))] * 3,
        out_specs=[_row_spec(BN)] * 3,
        out_shape=[jax.ShapeDtypeStruct((N, H), f32)] * 3,
    )
    vx, ux, vx2 = node_mm(
        x,
        eV_W.T, eV_b.reshape(1, H),
        nU_W.T, nU_b.reshape(1, H),
        nV_W.T, nV_b.reshape(1, H),
    )

    edge_tc = pl.pallas_call(
        _edge_body,
        grid=(EC // BE,),
        in_specs=[pl.BlockSpec((1, 1, BE), lambda i: (i, 0, 0),
                               memory_space=pltpu.SMEM)]
        + [_row_spec(BE)] * 4
        + [_full_spec((H, H))] + [_full_spec((1, H))] * 3,
        out_specs=[_row_spec(BE),
                   pl.BlockSpec((N, H), lambda i: (0, 0)),
                   pl.BlockSpec((N, H), lambda i: (0, 0))],
        out_shape=[jax.ShapeDtypeStruct((EC, H), f32),
                   jax.ShapeDtypeStruct((N, H), f32),
                   jax.ShapeDtypeStruct((N, H), f32)],
        scratch_shapes=[pltpu.VMEM((BE, H), f32)],
    )

    sc_gather = _make_sc_gather()
    gathered = [sc_gather(vx, vx2,
                          src[i * EC:(i + 1) * EC],
                          dst[i * EC:(i + 1) * EC])
                for i in range(NCK)]

    e_news, aggs = [], []
    for i in range(NCK):
        vxs, vxd, vx2s = gathered[i]
        e_new_i, agg_i, agg2_i = edge_tc(
            dst[i * EC:(i + 1) * EC].reshape(EC // BE, 1, BE),
            lax.slice_in_dim(e, i * EC, (i + 1) * EC),
            vxs, vxd, vx2s,
            eU_W.T, eU_b.reshape(1, H),
            ln_e_g.reshape(1, H), ln_e_b.reshape(1, H),
        )
        e_news.append(e_new_i)
        aggs += [agg_i, agg2_i]
    e_new = jnp.concatenate(e_news, axis=0)

    node_fin = pl.pallas_call(
        _node_fin_body,
        grid=(N // BN,),
        in_specs=[_row_spec(BN)] * 6 + [_full_spec((1, H))] * 2,
        out_specs=_row_spec(BN),
        out_shape=jax.ShapeDtypeStruct((N, H), f32),
    )
    x_new = node_fin(x, ux, *aggs,
                     ln_n_g.reshape(1, H), ln_n_b.reshape(1, H))

    return (x_new, e_new)


# speedup vs baseline: 1.0243x; 1.0243x over previous
import functools

import jax
import jax.numpy as jnp
from jax import lax
from jax.experimental import pallas as pl
from jax.experimental.pallas import tpu as pltpu
from jax.experimental.pallas import tpu_sc as plsc

N = 10000
E = 160000
H = 256

NC = 2
NS = 16
L = 16
NW = NC * NS

NCK = 2
EC = E // NCK
CH = 128
NCHUNK = EC // CH
CPW_G = -(-NCHUNK // NW)

@functools.cache
def _make_sc_gather():
    mesh = plsc.VectorSubcoreMesh(core_axis_name="c", subcore_axis_name="s")
    return functools.partial(
        pl.kernel,
        out_type=[jax.ShapeDtypeStruct((EC, H), jnp.float32)] * 3,
        mesh=mesh,
        scratch_types=[
            pltpu.VMEM((CH,), jnp.int32),
            pltpu.VMEM((CH,), jnp.int32),
            pltpu.VMEM((CH, H), jnp.float32),
            pltpu.VMEM((CH, H), jnp.float32),
            pltpu.VMEM((CH, H), jnp.float32),
            pltpu.SemaphoreType.DMA,
        ],
    )(_sc_gather_body)


def _sc_gather_body(vx_hbm, vx2_hbm, src_hbm, dst_hbm, oS, oD, o2,
                    si_v, di_v, bS, bD, b2, sem):
    wid = lax.axis_index("s") * NC + lax.axis_index("c")

    @pl.loop(0, CPW_G)
    def _(j):
        chunk = wid + j * NW

        @pl.when(chunk < NCHUNK)
        def _():
            base = chunk * CH
            pltpu.sync_copy(src_hbm.at[pl.ds(base, CH)], si_v)
            pltpu.sync_copy(dst_hbm.at[pl.ds(base, CH)], di_v)
            cS = pltpu.async_copy(vx_hbm.at[si_v], bS, sem)
            cD = pltpu.async_copy(vx_hbm.at[di_v], bD, sem)
            c2 = pltpu.async_copy(vx2_hbm.at[si_v], b2, sem)
            cS.wait()
            cD.wait()
            c2.wait()
            pltpu.sync_copy(bS, oS.at[pl.ds(base, CH)])
            pltpu.sync_copy(bD, oD.at[pl.ds(base, CH)])
            pltpu.sync_copy(b2, o2.at[pl.ds(base, CH)])


def _node_mm_body(x_ref, wv_ref, bv_ref, wu_ref, bu_ref, w2_ref, b2_ref,
                  vx_ref, ux_ref, vx2_ref):
    xb = x_ref[...]
    vx_ref[...] = jnp.dot(xb, wv_ref[...],
                          preferred_element_type=jnp.float32) + bv_ref[...]
    ux_ref[...] = jnp.dot(xb, wu_ref[...],
                          preferred_element_type=jnp.float32) + bu_ref[...]
    vx2_ref[...] = jnp.dot(xb, w2_ref[...],
                           preferred_element_type=jnp.float32) + b2_ref[...]


NACC = 4


def _edge_core(dst_ref, e_ref, vxs_ref, vxd_ref, vx2s_ref, w_ref, b_ref,
               g_ref, be_ref, enew_ref, accs, gn_scr):
    eb = e_ref[...]
    et = (jnp.dot(eb, w_ref[...], preferred_element_type=jnp.float32)
          + b_ref[...] + vxs_ref[...] + vxd_ref[...])
    gate = jax.nn.sigmoid(et)
    gn_scr[...] = vx2s_ref[...] * gate
    m = jnp.mean(et, axis=-1, keepdims=True)
    var = jnp.mean((et - m) * (et - m), axis=-1, keepdims=True)
    ln = (et - m) * lax.rsqrt(var + 1e-5) * g_ref[...] + be_ref[...]
    enew_ref[...] = eb + jnp.maximum(ln, 0.0)

    @pl.when(pl.program_id(0) == 0)
    def _():
        for ar in accs:
            ar[...] = jnp.zeros_like(ar)

    def _acc(kk, _):
        k = kk * NACC
        for t, ar in enumerate(accs):
            d = dst_ref[0, 0, k + t]
            ar[pl.ds(d, 1), :] += gn_scr[pl.ds(k + t, 1), :]
        return 0

    lax.fori_loop(0, BE // NACC, _acc, 0, unroll=2)


def _edge_body(dst_ref, e_ref, vxs_ref, vxd_ref, vx2s_ref, w_ref, b_ref,
               g_ref, be_ref, enew_ref, a0, a1, a2, a3, gn_scr):
    _edge_core(dst_ref, e_ref, vxs_ref, vxd_ref, vx2s_ref, w_ref, b_ref,
               g_ref, be_ref, enew_ref, (a0, a1, a2, a3), gn_scr)


def _edge_body_aliased(dst_ref, e_ref, vxs_ref, vxd_ref, vx2s_ref, w_ref,
                       b_ref, g_ref, be_ref, prev_ref, enew_ref,
                       a0, a1, a2, a3, gn_scr):
    del prev_ref
    _edge_core(dst_ref, e_ref, vxs_ref, vxd_ref, vx2s_ref, w_ref, b_ref,
               g_ref, be_ref, enew_ref, (a0, a1, a2, a3), gn_scr)


def _node_fin_body(x_ref, ux_ref, a0_ref, a1_ref, a2_ref, a3_ref,
                   a4_ref, a5_ref, a6_ref, a7_ref, g_ref, b_ref, xnew_ref):
    xt = (ux_ref[...]
          + ((a0_ref[...] + a1_ref[...]) + (a2_ref[...] + a3_ref[...]))
          + ((a4_ref[...] + a5_ref[...]) + (a6_ref[...] + a7_ref[...])))
    m = jnp.mean(xt, axis=-1, keepdims=True)
    var = jnp.mean((xt - m) * (xt - m), axis=-1, keepdims=True)
    ln = (xt - m) * lax.rsqrt(var + 1e-5) * g_ref[...] + b_ref[...]
    xnew_ref[...] = x_ref[...] + jnp.maximum(ln, 0.0)


def _row_spec(bm):
    return pl.BlockSpec((bm, H), lambda i: (i, 0))


def _full_spec(shape):
    return pl.BlockSpec(shape, lambda i: (0,) * len(shape))


BN = 2000
BE = 1000


def kernel(x, e, edge_index, eU_W, eU_b, eV_W, eV_b, nU_W, nU_b, nV_W, nV_b,
           ln_e_g, ln_e_b, ln_n_g, ln_n_b):
    f32 = jnp.float32
    src = edge_index[0]
    dst = edge_index[1]

    node_mm = pl.pallas_call(
        _node_mm_body,
        grid=(N // BN,),
        in_specs=[_row_spec(BN)] + [_full_spec((H, H)), _full_spec((1, H))] * 3,
        out_specs=[_row_spec(BN)] * 3,
        out_shape=[jax.ShapeDtypeStruct((N, H), f32)] * 3,
    )
    vx, ux, vx2 = node_mm(
        x,
        eV_W.T, eV_b.reshape(1, H),
        nU_W.T, nU_b.reshape(1, H),
        nV_W.T, nV_b.reshape(1, H),
    )

    grid_c = EC // BE
    acc_spec = [pl.BlockSpec((N, H), lambda i: (0, 0))] * NACC
    acc_shape = [jax.ShapeDtypeStruct((N, H), f32)] * NACC
    common_in = ([pl.BlockSpec((1, 1, BE), lambda i: (i, 0, 0),
                               memory_space=pltpu.SMEM)]
                 + [_row_spec(BE)] * 4
                 + [_full_spec((H, H))] + [_full_spec((1, H))] * 3)

    edge_tc0 = pl.pallas_call(
        _edge_body,
        grid=(grid_c,),
        in_specs=common_in,
        out_specs=[_row_spec(BE)] + acc_spec,
        out_shape=[jax.ShapeDtypeStruct((E, H), f32)] + acc_shape,
        scratch_shapes=[pltpu.VMEM((BE, H), f32)],
    )
    edge_tc1 = pl.pallas_call(
        _edge_body_aliased,
        grid=(grid_c,),
        in_specs=common_in + [pl.BlockSpec((8, 128), lambda i: (0, 0))],
        out_specs=[pl.BlockSpec((BE, H), lambda i: (i + grid_c, 0))]
        + acc_spec,
        out_shape=[jax.ShapeDtypeStruct((E, H), f32)] + acc_shape,
        scratch_shapes=[pltpu.VMEM((BE, H), f32)],
        input_output_aliases={9: 0},
    )

    sc_gather = _make_sc_gather()
    gathered = [sc_gather(vx, vx2,
                          src[i * EC:(i + 1) * EC],
                          dst[i * EC:(i + 1) * EC])
                for i in range(NCK)]

    wargs0 = (eU_W.T, eU_b.reshape(1, H),
              ln_e_g.reshape(1, H), ln_e_b.reshape(1, H))
    e_new, *aggs = edge_tc0(
        dst[:EC].reshape(grid_c, 1, BE),
        lax.slice_in_dim(e, 0, EC),
        *gathered[0], *wargs0,
    )
    e_new, *aggs1 = edge_tc1(
        dst[EC:].reshape(grid_c, 1, BE),
        lax.slice_in_dim(e, EC, E),
        *gathered[1], *wargs0,
        e_new,
    )
    aggs += aggs1

    node_fin = pl.pallas_call(
        _node_fin_body,
        grid=(N // BN,),
        in_specs=[_row_spec(BN)] * 10 + [_full_spec((1, H))] * 2,
        out_specs=_row_spec(BN),
        out_shape=jax.ShapeDtypeStruct((N, H), f32),
    )
    x_new = node_fin(x, ux, *aggs,
                     ln_n_g.reshape(1, H), ln_n_b.reshape(1, H))

    return (x_new, e_new)


# speedup vs baseline: 1.1106x; 1.0843x over previous
import functools

import jax
import jax.numpy as jnp
from jax import lax
from jax.experimental import pallas as pl
from jax.experimental.pallas import tpu as pltpu
from jax.experimental.pallas import tpu_sc as plsc

N = 10000
E = 160000
H = 256

NC = 2
NS = 16
L = 16
NW = NC * NS

NCK = 2
EC = E // NCK
CH = 128
NCHUNK = EC // CH
CPW_G = -(-NCHUNK // NW)

@functools.cache
def _make_sc_gather():
    mesh = plsc.VectorSubcoreMesh(core_axis_name="c", subcore_axis_name="s")
    return functools.partial(
        pl.kernel,
        out_type=[jax.ShapeDtypeStruct((EC, H), jnp.float32)] * 3,
        mesh=mesh,
        scratch_types=[
            pltpu.VMEM((CH,), jnp.int32),
            pltpu.VMEM((CH,), jnp.int32),
            pltpu.VMEM((CH, H), jnp.float32),
            pltpu.VMEM((CH, H), jnp.float32),
            pltpu.VMEM((CH, H), jnp.float32),
            pltpu.SemaphoreType.DMA,
        ],
    )(_sc_gather_body)


def _sc_gather_body(vx_hbm, vx2_hbm, src_hbm, dst_hbm, oS, oD, o2,
                    si_v, di_v, bS, bD, b2, sem):
    wid = lax.axis_index("s") * NC + lax.axis_index("c")

    @pl.loop(0, CPW_G)
    def _(j):
        chunk = wid + j * NW

        @pl.when(chunk < NCHUNK)
        def _():
            base = chunk * CH
            pltpu.sync_copy(src_hbm.at[pl.ds(base, CH)], si_v)
            pltpu.sync_copy(dst_hbm.at[pl.ds(base, CH)], di_v)
            cS = pltpu.async_copy(vx_hbm.at[si_v], bS, sem)
            cD = pltpu.async_copy(vx_hbm.at[di_v], bD, sem)
            c2 = pltpu.async_copy(vx2_hbm.at[si_v], b2, sem)
            cS.wait()
            cD.wait()
            c2.wait()
            pltpu.sync_copy(bS, oS.at[pl.ds(base, CH)])
            pltpu.sync_copy(bD, oD.at[pl.ds(base, CH)])
            pltpu.sync_copy(b2, o2.at[pl.ds(base, CH)])


def _node_mm_body(x_ref, wv_ref, bv_ref, wu_ref, bu_ref, w2_ref, b2_ref,
                  vx_ref, ux_ref, vx2_ref):
    xb = x_ref[...]
    vx_ref[...] = jnp.dot(xb, wv_ref[...],
                          preferred_element_type=jnp.float32) + bv_ref[...]
    ux_ref[...] = jnp.dot(xb, wu_ref[...],
                          preferred_element_type=jnp.float32) + bu_ref[...]
    vx2_ref[...] = jnp.dot(xb, w2_ref[...],
                           preferred_element_type=jnp.float32) + b2_ref[...]


NACC = 4


def _edge_core(dst_ref, e_ref, vxs_ref, vxd_ref, vx2s_ref, w_ref, b_ref,
               g_ref, be_ref, enew_ref, accs, gn_scr):
    eb = e_ref[...]
    et = (jnp.dot(eb, w_ref[...], preferred_element_type=jnp.float32)
          + b_ref[...] + vxs_ref[...] + vxd_ref[...])
    gate = jax.nn.sigmoid(et)
    gn_scr[...] = vx2s_ref[...] * gate
    m = jnp.mean(et, axis=-1, keepdims=True)
    var = jnp.mean((et - m) * (et - m), axis=-1, keepdims=True)
    ln = (et - m) * lax.rsqrt(var + 1e-5) * g_ref[...] + be_ref[...]
    enew_ref[...] = eb + jnp.maximum(ln, 0.0)

    @pl.when(pl.program_id(0) == 0)
    def _():
        for ar in accs:
            ar[...] = jnp.zeros_like(ar)

    def _acc(kk, _):
        k = kk * NACC
        for t, ar in enumerate(accs):
            d = dst_ref[0, 0, k + t]
            ar[pl.ds(d, 1), :] += gn_scr[pl.ds(k + t, 1), :]
        return 0

    lax.fori_loop(0, BE // NACC, _acc, 0, unroll=2)


def _edge_body(dst_ref, e_ref, vxs_ref, vxd_ref, vx2s_ref, w_ref, b_ref,
               g_ref, be_ref, enew_ref, a0, a1, a2, a3, gn_scr):
    _edge_core(dst_ref, e_ref, vxs_ref, vxd_ref, vx2s_ref, w_ref, b_ref,
               g_ref, be_ref, enew_ref, (a0, a1, a2, a3), gn_scr)


def _edge_body_aliased(dst_ref, e_ref, vxs_ref, vxd_ref, vx2s_ref, w_ref,
                       b_ref, g_ref, be_ref, prev_ref, enew_ref,
                       a0, a1, a2, a3, gn_scr):
    del prev_ref
    _edge_core(dst_ref, e_ref, vxs_ref, vxd_ref, vx2s_ref, w_ref, b_ref,
               g_ref, be_ref, enew_ref, (a0, a1, a2, a3), gn_scr)


def _node_fin_body(x_ref, ux_ref, a0_ref, a1_ref, a2_ref, a3_ref,
                   a4_ref, a5_ref, a6_ref, a7_ref, g_ref, b_ref, xnew_ref):
    xt = (ux_ref[...]
          + ((a0_ref[...] + a1_ref[...]) + (a2_ref[...] + a3_ref[...]))
          + ((a4_ref[...] + a5_ref[...]) + (a6_ref[...] + a7_ref[...])))
    m = jnp.mean(xt, axis=-1, keepdims=True)
    var = jnp.mean((xt - m) * (xt - m), axis=-1, keepdims=True)
    ln = (xt - m) * lax.rsqrt(var + 1e-5) * g_ref[...] + b_ref[...]
    xnew_ref[...] = x_ref[...] + jnp.maximum(ln, 0.0)


def _row_spec(bm):
    return pl.BlockSpec((bm, H), lambda i: (i, 0))


def _full_spec(shape):
    return pl.BlockSpec(shape, lambda i: (0,) * len(shape))


BN = 2000
BE = 1000


def kernel(x, e, edge_index, eU_W, eU_b, eV_W, eV_b, nU_W, nU_b, nV_W, nV_b,
           ln_e_g, ln_e_b, ln_n_g, ln_n_b):
    f32 = jnp.float32
    src = edge_index[0]
    dst = edge_index[1]

    node_mm = pl.pallas_call(
        _node_mm_body,
        grid=(N // BN,),
        in_specs=[_row_spec(BN)] + [_full_spec((H, H)), _full_spec((1, H))] * 3,
        out_specs=[_row_spec(BN)] * 3,
        out_shape=[jax.ShapeDtypeStruct((N, H), f32)] * 3,
    )
    vx, ux, vx2 = node_mm(
        x,
        eV_W.T, eV_b.reshape(1, H),
        nU_W.T, nU_b.reshape(1, H),
        nV_W.T, nV_b.reshape(1, H),
    )

    grid_c = EC // BE
    acc_spec = [pl.BlockSpec((N, H), lambda i: (0, 0))] * NACC
    acc_shape = [jax.ShapeDtypeStruct((N, H), f32)] * NACC

    def _chunk_in(ci):
        return ([pl.BlockSpec((1, 1, BE), lambda i: (i + ci * grid_c, 0, 0),
                              memory_space=pltpu.SMEM),
                 pl.BlockSpec((BE, H), lambda i: (i + ci * grid_c, 0))]
                + [_row_spec(BE)] * 3
                + [_full_spec((H, H))] + [_full_spec((1, H))] * 3)

    edge_tc0 = pl.pallas_call(
        _edge_body,
        grid=(grid_c,),
        in_specs=_chunk_in(0),
        out_specs=[_row_spec(BE)] + acc_spec,
        out_shape=[jax.ShapeDtypeStruct((E, H), f32)] + acc_shape,
        scratch_shapes=[pltpu.VMEM((BE, H), f32)],
    )
    edge_tc1 = pl.pallas_call(
        _edge_body_aliased,
        grid=(grid_c,),
        in_specs=_chunk_in(1) + [pl.BlockSpec((8, 128), lambda i: (0, 0))],
        out_specs=[pl.BlockSpec((BE, H), lambda i: (i + grid_c, 0))]
        + acc_spec,
        out_shape=[jax.ShapeDtypeStruct((E, H), f32)] + acc_shape,
        scratch_shapes=[pltpu.VMEM((BE, H), f32)],
        input_output_aliases={9: 0},
    )

    sc_gather = _make_sc_gather()
    gathered = [sc_gather(vx, vx2,
                          src[i * EC:(i + 1) * EC],
                          dst[i * EC:(i + 1) * EC])
                for i in range(NCK)]

    dst3 = dst.reshape(E // BE, 1, BE)
    wargs0 = (eU_W.T, eU_b.reshape(1, H),
              ln_e_g.reshape(1, H), ln_e_b.reshape(1, H))
    e_new, *aggs = edge_tc0(dst3, e, *gathered[0], *wargs0)
    e_new, *aggs1 = edge_tc1(dst3, e, *gathered[1], *wargs0, e_new)
    aggs += aggs1

    node_fin = pl.pallas_call(
        _node_fin_body,
        grid=(N // BN,),
        in_specs=[_row_spec(BN)] * 10 + [_full_spec((1, H))] * 2,
        out_specs=_row_spec(BN),
        out_shape=jax.ShapeDtypeStruct((N, H), f32),
    )
    x_new = node_fin(x, ux, *aggs,
                     ln_n_g.reshape(1, H), ln_n_b.reshape(1, H))

    return (x_new, e_new)


# speedup vs baseline: 1.2497x; 1.1253x over previous
import functools

import jax
import jax.numpy as jnp
from jax import lax
from jax.experimental import pallas as pl
from jax.experimental.pallas import tpu as pltpu
from jax.experimental.pallas import tpu_sc as plsc

N = 10000
E = 160000
H = 256

NC = 2
NS = 16
L = 16
NW = NC * NS

NCK = 5
EC = E // NCK
CH = 128
NCHUNK = EC // CH
CPW_G = -(-NCHUNK // NW)

@functools.cache
def _make_sc_gather():
    mesh = plsc.VectorSubcoreMesh(core_axis_name="c", subcore_axis_name="s")
    return functools.partial(
        pl.kernel,
        out_type=[jax.ShapeDtypeStruct((EC, H), jnp.float32)] * 3,
        mesh=mesh,
        scratch_types=[
            pltpu.VMEM((CH,), jnp.int32),
            pltpu.VMEM((CH,), jnp.int32),
            pltpu.VMEM((CH, H), jnp.float32),
            pltpu.VMEM((CH, H), jnp.float32),
            pltpu.VMEM((CH, H), jnp.float32),
            pltpu.SemaphoreType.DMA,
        ],
    )(_sc_gather_body)


def _sc_gather_body(vx_hbm, vx2_hbm, src_hbm, dst_hbm, oS, oD, o2,
                    si_v, di_v, bS, bD, b2, sem):
    wid = lax.axis_index("s") * NC + lax.axis_index("c")

    @pl.loop(0, CPW_G)
    def _(j):
        chunk = wid + j * NW

        @pl.when(chunk < NCHUNK)
        def _():
            base = chunk * CH
            pltpu.sync_copy(src_hbm.at[pl.ds(base, CH)], si_v)
            pltpu.sync_copy(dst_hbm.at[pl.ds(base, CH)], di_v)
            cS = pltpu.async_copy(vx_hbm.at[si_v], bS, sem)
            cD = pltpu.async_copy(vx_hbm.at[di_v], bD, sem)
            c2 = pltpu.async_copy(vx2_hbm.at[si_v], b2, sem)
            cS.wait()
            cD.wait()
            c2.wait()
            pltpu.sync_copy(bS, oS.at[pl.ds(base, CH)])
            pltpu.sync_copy(bD, oD.at[pl.ds(base, CH)])
            pltpu.sync_copy(b2, o2.at[pl.ds(base, CH)])


def _node_mm_body(x_ref, wv_ref, bv_ref, wu_ref, bu_ref, w2_ref, b2_ref,
                  vx_ref, ux_ref, vx2_ref):
    xb = x_ref[...]
    vx_ref[...] = jnp.dot(xb, wv_ref[...],
                          preferred_element_type=jnp.float32) + bv_ref[...]
    ux_ref[...] = jnp.dot(xb, wu_ref[...],
                          preferred_element_type=jnp.float32) + bu_ref[...]
    vx2_ref[...] = jnp.dot(xb, w2_ref[...],
                           preferred_element_type=jnp.float32) + b2_ref[...]


NACC = 2


def _edge_core(dst_ref, e_ref, vxs_ref, vxd_ref, vx2s_ref, w_ref, b_ref,
               g_ref, be_ref, enew_ref, accs, gn_scr):
    eb = e_ref[...]
    et = (jnp.dot(eb, w_ref[...], preferred_element_type=jnp.float32)
          + b_ref[...] + vxs_ref[...] + vxd_ref[...])
    gate = jax.nn.sigmoid(et)
    gn_scr[...] = vx2s_ref[...] * gate
    m = jnp.mean(et, axis=-1, keepdims=True)
    var = jnp.mean((et - m) * (et - m), axis=-1, keepdims=True)
    ln = (et - m) * lax.rsqrt(var + 1e-5) * g_ref[...] + be_ref[...]
    enew_ref[...] = eb + jnp.maximum(ln, 0.0)

    @pl.when(pl.program_id(0) == 0)
    def _():
        for ar in accs:
            ar[...] = jnp.zeros_like(ar)

    def _acc(kk, _):
        k = kk * NACC
        for t, ar in enumerate(accs):
            d = dst_ref[0, 0, k + t]
            ar[pl.ds(d, 1), :] += gn_scr[pl.ds(k + t, 1), :]
        return 0

    lax.fori_loop(0, BE // NACC, _acc, 0, unroll=4)


def _edge_body(dst_ref, e_ref, vxs_ref, vxd_ref, vx2s_ref, w_ref, b_ref,
               g_ref, be_ref, enew_ref, a0, a1, gn_scr):
    _edge_core(dst_ref, e_ref, vxs_ref, vxd_ref, vx2s_ref, w_ref, b_ref,
               g_ref, be_ref, enew_ref, (a0, a1), gn_scr)


def _edge_body_aliased(dst_ref, e_ref, vxs_ref, vxd_ref, vx2s_ref, w_ref,
                       b_ref, g_ref, be_ref, prev_ref, enew_ref,
                       a0, a1, gn_scr):
    del prev_ref
    _edge_core(dst_ref, e_ref, vxs_ref, vxd_ref, vx2s_ref, w_ref, b_ref,
               g_ref, be_ref, enew_ref, (a0, a1), gn_scr)


def _node_fin_body(x_ref, ux_ref, *rest):
    agg_refs = rest[:-3]
    g_ref, b_ref, xnew_ref = rest[-3:]
    xt = ux_ref[...]
    for ar in agg_refs:
        xt = xt + ar[...]
    m = jnp.mean(xt, axis=-1, keepdims=True)
    var = jnp.mean((xt - m) * (xt - m), axis=-1, keepdims=True)
    ln = (xt - m) * lax.rsqrt(var + 1e-5) * g_ref[...] + b_ref[...]
    xnew_ref[...] = x_ref[...] + jnp.maximum(ln, 0.0)


def _row_spec(bm):
    return pl.BlockSpec((bm, H), lambda i: (i, 0))


def _full_spec(shape):
    return pl.BlockSpec(shape, lambda i: (0,) * len(shape))


BN = 2000
BE = 2000


def kernel(x, e, edge_index, eU_W, eU_b, eV_W, eV_b, nU_W, nU_b, nV_W, nV_b,
           ln_e_g, ln_e_b, ln_n_g, ln_n_b):
    f32 = jnp.float32
    src = edge_index[0]
    dst = edge_index[1]

    node_mm = pl.pallas_call(
        _node_mm_body,
        grid=(N // BN,),
        in_specs=[_row_spec(BN)] + [_full_spec((H, H)), _full_spec((1, H))] * 3,
        out_specs=[_row_spec(BN)] * 3,
        out_shape=[jax.ShapeDtypeStruct((N, H), f32)] * 3,
    )
    vx, ux, vx2 = node_mm(
        x,
        eV_W.T, eV_b.reshape(1, H),
        nU_W.T, nU_b.reshape(1, H),
        nV_W.T, nV_b.reshape(1, H),
    )

    grid_c = EC // BE
    acc_spec = [pl.BlockSpec((N, H), lambda i: (0, 0))] * NACC
    acc_shape = [jax.ShapeDtypeStruct((N, H), f32)] * NACC

    def _chunk_in(ci):
        return ([pl.BlockSpec((1, 1, BE),
                              lambda i, ci=ci: (i + ci * grid_c, 0, 0),
                              memory_space=pltpu.SMEM),
                 pl.BlockSpec((BE, H),
                              lambda i, ci=ci: (i + ci * grid_c, 0))]
                + [_row_spec(BE)] * 3
                + [_full_spec((H, H))] + [_full_spec((1, H))] * 3)

    def _edge_call(ci):
        out0 = pl.BlockSpec((BE, H), lambda i, ci=ci: (i + ci * grid_c, 0))
        if ci == 0:
            return pl.pallas_call(
                _edge_body,
                grid=(grid_c,),
                in_specs=_chunk_in(0),
                out_specs=[out0] + acc_spec,
                out_shape=[jax.ShapeDtypeStruct((E, H), f32)] + acc_shape,
                scratch_shapes=[pltpu.VMEM((BE, H), f32)],
            )
        return pl.pallas_call(
            _edge_body_aliased,
            grid=(grid_c,),
            in_specs=_chunk_in(ci) + [pl.BlockSpec((8, 128),
                                                   lambda i: (0, 0))],
            out_specs=[out0] + acc_spec,
            out_shape=[jax.ShapeDtypeStruct((E, H), f32)] + acc_shape,
            scratch_shapes=[pltpu.VMEM((BE, H), f32)],
            input_output_aliases={9: 0},
        )

    sc_gather = _make_sc_gather()
    gathered = [sc_gather(vx, vx2,
                          src[i * EC:(i + 1) * EC],
                          dst[i * EC:(i + 1) * EC])
                for i in range(NCK)]

    dst3 = dst.reshape(E // BE, 1, BE)
    wargs0 = (eU_W.T, eU_b.reshape(1, H),
              ln_e_g.reshape(1, H), ln_e_b.reshape(1, H))
    e_new, *aggs = _edge_call(0)(dst3, e, *gathered[0], *wargs0)
    for ci in range(1, NCK):
        e_new, *aggs_i = _edge_call(ci)(dst3, e, *gathered[ci], *wargs0,
                                        e_new)
        aggs += aggs_i

    node_fin = pl.pallas_call(
        _node_fin_body,
        grid=(N // BN,),
        in_specs=[_row_spec(BN)] * (2 + NCK * NACC) + [_full_spec((1, H))] * 2,
        out_specs=_row_spec(BN),
        out_shape=jax.ShapeDtypeStruct((N, H), f32),
    )
    x_new = node_fin(x, ux, *aggs,
                     ln_n_g.reshape(1, H), ln_n_b.reshape(1, H))

    return (x_new, e_new)
